# baseline (speedup 1.0000x reference)
import functools

import jax
import jax.numpy as jnp
import numpy as np
from jax.experimental import pallas as pl
from jax.experimental.pallas import tpu as pltpu

B, S, D, H = 4, 4096, 1024, 16
DH = D // H
BUCKET = 64
NH = 4
NCH = S // BUCKET
NB = NCH
DFF = 4096

BM = 512


def _matmul_body(x_ref, w_ref, o_ref):
    o_ref[...] = jnp.dot(x_ref[...], w_ref[...],
                         preferred_element_type=jnp.float32)


def _pallas_matmul(x, w):
    M, K = x.shape
    N = w.shape[1]
    return pl.pallas_call(
        _matmul_body,
        grid=(M // BM,),
        in_specs=[
            pl.BlockSpec((BM, K), lambda i: (i, 0)),
            pl.BlockSpec((K, N), lambda i: (0, 0)),
        ],
        out_specs=pl.BlockSpec((BM, N), lambda i: (i, 0)),
        out_shape=jax.ShapeDtypeStruct((M, N), jnp.float32),
    )(x, w)


def _ffn_body(x_ref, w1_ref, b1_ref, w2_ref, b2_ref, o_ref):
    h = jnp.dot(x_ref[...], w1_ref[...], preferred_element_type=jnp.float32)
    h = jnp.maximum(h + b1_ref[...], 0.0)
    o_ref[...] = (jnp.dot(h, w2_ref[...], preferred_element_type=jnp.float32)
                  + b2_ref[...])


def _pallas_ffn(x, w1, b1, w2, b2):
    M = x.shape[0]
    return pl.pallas_call(
        _ffn_body,
        grid=(M // BM,),
        in_specs=[
            pl.BlockSpec((BM, D), lambda i: (i, 0)),
            pl.BlockSpec((D, DFF), lambda i: (0, 0)),
            pl.BlockSpec((1, DFF), lambda i: (0, 0)),
            pl.BlockSpec((DFF, D), lambda i: (0, 0)),
            pl.BlockSpec((1, D), lambda i: (0, 0)),
        ],
        out_specs=pl.BlockSpec((BM, D), lambda i: (i, 0)),
        out_shape=jax.ShapeDtypeStruct((M, D), jnp.float32),
    )(x, w1, b1.reshape(1, DFF), w2, b2.reshape(1, D))


def _lsh_attention(src, Wqk, Wv, Wo, rot):
    x2 = src.reshape(B * S, D)
    qk = _pallas_matmul(x2, Wqk).reshape(B, S, H, DH).transpose(0, 2, 1, 3)
    v = _pallas_matmul(x2, Wv).reshape(B, S, H, DH).transpose(0, 2, 1, 3)
    qn = qk / (jnp.linalg.norm(qk, axis=-1, keepdims=True) + 1e-6)
    rotated = jnp.einsum('bhsd,dnr->bhsnr', qn, rot)
    buckets = jnp.argmax(
        jnp.concatenate([rotated, -rotated], axis=-1), axis=-1)
    pos = jnp.arange(S)
    scale = 1.0 / np.sqrt(DH)
    eye = jnp.eye(BUCKET, dtype=bool)
    self_mask = jnp.concatenate(
        [eye, jnp.zeros((BUCKET, BUCKET), bool)], axis=-1)
    outs, lses = [], []
    for r in range(NH):
        b_r = buckets[..., r]
        skey = b_r * S + pos
        order = jnp.argsort(skey, axis=-1)
        inv = jnp.argsort(order, axis=-1)
        qk_s = jnp.take_along_axis(qk, order[..., None], axis=2)
        v_s = jnp.take_along_axis(v, order[..., None], axis=2)
        qc = qk_s.reshape(B, H, NCH, BUCKET, DH)
        kc = qc / (jnp.linalg.norm(qc, axis=-1, keepdims=True) + 1e-6)
        vc = v_s.reshape(B, H, NCH, BUCKET, DH)
        kcat = jnp.concatenate([kc, jnp.roll(kc, 1, axis=2)], axis=3)
        vcat = jnp.concatenate([vc, jnp.roll(vc, 1, axis=2)], axis=3)
        dots = jnp.einsum('bhcqd,bhckd->bhcqk', qc, kcat) * scale
        dots = jnp.where(self_mask, -1e5, dots)
        lse = jax.nn.logsumexp(dots, axis=-1)
        probs = jnp.exp(dots - lse[..., None])
        oc = jnp.einsum('bhcqk,bhckd->bhcqd', probs, vcat)
        o = oc.reshape(B, H, S, DH)
        l = lse.reshape(B, H, S)
        o = jnp.take_along_axis(o, inv[..., None], axis=2)
        l = jnp.take_along_axis(l, inv, axis=2)
        outs.append(o)
        lses.append(l)
    o_stack = jnp.stack(outs, 0)
    l_stack = jnp.stack(lses, 0)
    w = jax.nn.softmax(l_stack, axis=0)[..., None]
    attn = jnp.sum(o_stack * w, axis=0)
    attn = attn.transpose(0, 2, 1, 3).reshape(B * S, D)
    return _pallas_matmul(attn, Wo)


@jax.jit
def kernel(src, Wqk, Wv, Wo, rot, W1, b1, W2, b2):
    a = _lsh_attention(src, Wqk, Wv, Wo, rot)
    out = _pallas_ffn(a, W1, b1, W2, b2)
    return out.reshape(B, S, D)


# speedup vs baseline: 2.1797x; 2.1797x over previous
import jax
import jax.numpy as jnp
import numpy as np
from jax import lax
from jax.experimental import pallas as pl
from jax.experimental.pallas import tpu as pltpu

B, S, D, H = 4, 4096, 1024, 16
DH = D // H
BUCKET = 64
NH = 4
NCH = S // BUCKET
DFF = 4096
NINST = B * H * NH
SCALE = 1.0 / np.sqrt(DH)

BM = 512


def _k1_body(src_ref, wqk_ref, wv_ref, rot_ref, qk_ref, v_ref, bkt_ref):
    x = src_ref[0]
    qk = jnp.dot(x, wqk_ref[...], preferred_element_type=jnp.float32)
    v = jnp.dot(x, wv_ref[...], preferred_element_type=jnp.float32)
    qk_ref[0] = qk
    v_ref[0] = v
    rot2 = rot_ref[...]
    iota = lax.broadcasted_iota(jnp.int32, (BM, 2 * 32), 1)
    parts = []
    for h in range(H):
        qh = qk[:, h * DH:(h + 1) * DH]
        n = jnp.sqrt(jnp.sum(qh * qh, axis=1, keepdims=True))
        qn = qh / (n + 1e-6)
        rh = jnp.dot(qn, rot2, preferred_element_type=jnp.float32)
        for r in range(NH):
            vals = rh[:, r * 32:(r + 1) * 32]
            cat = jnp.concatenate([vals, -vals], axis=1)
            m = jnp.max(cat, axis=1, keepdims=True)
            idx = jnp.min(jnp.where(cat >= m, iota, 2 * 32), axis=1,
                          keepdims=True)
            parts.append(idx)
    bkt_ref[0] = jnp.concatenate(parts, axis=1)


def k1_proj_hash(src, Wqk, Wv, rot):
    rot2 = rot.reshape(DH, NH * 32)
    return pl.pallas_call(
        _k1_body,
        grid=(B, S // BM),
        in_specs=[
            pl.BlockSpec((1, BM, D), lambda b, s: (b, s, 0)),
            pl.BlockSpec((D, D), lambda b, s: (0, 0)),
            pl.BlockSpec((D, D), lambda b, s: (0, 0)),
            pl.BlockSpec((DH, NH * 32), lambda b, s: (0, 0)),
        ],
        out_specs=[
            pl.BlockSpec((1, BM, D), lambda b, s: (b, s, 0)),
            pl.BlockSpec((1, BM, D), lambda b, s: (b, s, 0)),
            pl.BlockSpec((1, BM, H * NH), lambda b, s: (b, s, 0)),
        ],
        out_shape=[
            jax.ShapeDtypeStruct((B, S, D), jnp.float32),
            jax.ShapeDtypeStruct((B, S, D), jnp.float32),
            jax.ShapeDtypeStruct((B, S, H * NH), jnp.int32),
        ],
    )(src, Wqk, Wv, rot2)


TS = 128
NT = S // TS


def _k2_body(bkt_ref, inv_ref):
    bkt = bkt_ref[0]
    totals = []
    for beta in range(BUCKET):
        totals.append(jnp.sum((bkt == beta).astype(jnp.float32), axis=0,
                              keepdims=True))
    offs = []
    run = jnp.zeros((1, H * NH), jnp.float32)
    for beta in range(BUCKET):
        offs.append(run)
        run = run + totals[beta]
    ii = lax.broadcasted_iota(jnp.int32, (TS, TS), 0)
    jj = lax.broadcasted_iota(jnp.int32, (TS, TS), 1)
    T = (jj <= ii).astype(jnp.float32)

    def tile_step(t, carrys):
        blk = bkt_ref[0, pl.ds(t * TS, TS), :]
        acc = jnp.zeros((TS, H * NH), jnp.float32)
        new_carrys = []
        for beta in range(BUCKET):
            I = (blk == beta).astype(jnp.float32)
            incl = jnp.dot(T, I, preferred_element_type=jnp.float32)
            acc = acc + I * (incl - 1.0 + carrys[beta] + offs[beta])
            new_carrys.append(carrys[beta] + incl[TS - 1:TS, :])
        inv_ref[0, pl.ds(t * TS, TS), :] = acc.astype(jnp.int32)
        return tuple(new_carrys)

    lax.fori_loop(0, NT, tile_step,
                  tuple(jnp.zeros((1, H * NH), jnp.float32)
                        for _ in range(BUCKET)))


def k2_inv(bkt):
    return pl.pallas_call(
        _k2_body,
        grid=(B,),
        in_specs=[pl.BlockSpec((1, S, H * NH), lambda b: (b, 0, 0))],
        out_specs=pl.BlockSpec((1, S, H * NH), lambda b: (b, 0, 0)),
        out_shape=jax.ShapeDtypeStruct((B, S, H * NH), jnp.int32),
    )(bkt)


def _k4_body(qk_ref, v_ref, o_ref, lse_ref, kn_ref):
    qk = qk_ref[0]
    n = jnp.sqrt(jnp.sum(qk * qk, axis=1, keepdims=True))
    kn_ref[...] = qk * (1.0 / (n + 1e-6))
    qi = lax.broadcasted_iota(jnp.int32, (BUCKET, 2 * BUCKET), 0)
    ki = lax.broadcasted_iota(jnp.int32, (BUCKET, 2 * BUCKET), 1)
    selfmask = qi == ki

    def chunk(c, _):
        p = (c + NCH - 1) % NCH
        qc = qk_ref[0, pl.ds(c * BUCKET, BUCKET), :]
        kcat = jnp.concatenate(
            [kn_ref[pl.ds(c * BUCKET, BUCKET), :],
             kn_ref[pl.ds(p * BUCKET, BUCKET), :]], axis=0)
        vcat = jnp.concatenate(
            [v_ref[0, pl.ds(c * BUCKET, BUCKET), :],
             v_ref[0, pl.ds(p * BUCKET, BUCKET), :]], axis=0)
        dots = lax.dot_general(qc, kcat, (((1,), (1,)), ((), ())),
                               preferred_element_type=jnp.float32) * SCALE
        dots = jnp.where(selfmask, -1e5, dots)
        m = jnp.max(dots, axis=1, keepdims=True)
        p_ = jnp.exp(dots - m)
        s = jnp.sum(p_, axis=1, keepdims=True)
        lse = m + jnp.log(s)
        o = jnp.dot(p_, vcat, preferred_element_type=jnp.float32) * (1.0 / s)
        o_ref[0, pl.ds(c * BUCKET, BUCKET), :] = o
        lse_ref[0, pl.ds(c * BUCKET, BUCKET), :] = jnp.broadcast_to(
            lse, (BUCKET, 16))
        return 0

    lax.fori_loop(0, NCH, chunk, 0)


def k4_attention(qk_s, v_s):
    return pl.pallas_call(
        _k4_body,
        grid=(NINST,),
        in_specs=[
            pl.BlockSpec((1, S, DH), lambda i: (i, 0, 0)),
            pl.BlockSpec((1, S, DH), lambda i: (i, 0, 0)),
        ],
        out_specs=[
            pl.BlockSpec((1, S, DH), lambda i: (i, 0, 0)),
            pl.BlockSpec((1, S, 16), lambda i: (i, 0, 0)),
        ],
        out_shape=[
            jax.ShapeDtypeStruct((NINST, S, DH), jnp.float32),
            jax.ShapeDtypeStruct((NINST, S, 16), jnp.float32),
        ],
        scratch_shapes=[pltpu.VMEM((S, DH), jnp.float32)],
    )(qk_s, v_s)


BMC = 128


def _k6_body(o_ref, lse_ref, wo_ref, out_ref):
    ls = [lse_ref[r, 0] for r in range(NH)]
    m = ls[0]
    for r in range(1, NH):
        m = jnp.maximum(m, ls[r])
    es = [jnp.exp(l - m) for l in ls]
    ssum = es[0]
    for r in range(1, NH):
        ssum = ssum + es[r]
    bmat = jnp.full((16, DH), 1.0 / 16.0, jnp.float32)
    parts = []
    for h in range(H):
        acc = jnp.zeros((BMC, DH), jnp.float32)
        for r in range(NH):
            w = jnp.dot(es[r][h] * (1.0 / ssum[h]), bmat,
                        preferred_element_type=jnp.float32)
            acc = acc + o_ref[r, 0, :, h * DH:(h + 1) * DH] * w
        parts.append(acc)
    attn = jnp.concatenate(parts, axis=1)
    out_ref[0] = jnp.dot(attn, wo_ref[...], preferred_element_type=jnp.float32)


def k6_combine_wo(o_u, lse_u, Wo):
    return pl.pallas_call(
        _k6_body,
        grid=(B, S // BMC),
        in_specs=[
            pl.BlockSpec((NH, 1, BMC, D), lambda b, s: (0, b, s, 0)),
            pl.BlockSpec((NH, 1, H, BMC, 16), lambda b, s: (0, b, 0, s, 0)),
            pl.BlockSpec((D, D), lambda b, s: (0, 0)),
        ],
        out_specs=pl.BlockSpec((1, BMC, D), lambda b, s: (b, s, 0)),
        out_shape=jax.ShapeDtypeStruct((B, S, D), jnp.float32),
    )(o_u, lse_u, Wo)


FT = 1024


def _k7_body(x_ref, w1_ref, b1_ref, w2_ref, b2_ref, o_ref):
    t = pl.program_id(2)
    h = jnp.dot(x_ref[0], w1_ref[...], preferred_element_type=jnp.float32)
    h = jnp.maximum(h + b1_ref[...], 0.0)
    part = jnp.dot(h, w2_ref[...], preferred_element_type=jnp.float32)

    @pl.when(t == 0)
    def _():
        o_ref[0] = part + b2_ref[...]

    @pl.when(t != 0)
    def _():
        o_ref[0] = o_ref[0] + part


def k7_ffn(x, w1, b1, w2, b2):
    return pl.pallas_call(
        _k7_body,
        grid=(B, S // BM, DFF // FT),
        in_specs=[
            pl.BlockSpec((1, BM, D), lambda b, s, t: (b, s, 0)),
            pl.BlockSpec((D, FT), lambda b, s, t: (0, t)),
            pl.BlockSpec((1, FT), lambda b, s, t: (0, t)),
            pl.BlockSpec((FT, D), lambda b, s, t: (t, 0)),
            pl.BlockSpec((1, D), lambda b, s, t: (0, 0)),
        ],
        out_specs=pl.BlockSpec((1, BM, D), lambda b, s, t: (b, s, 0)),
        out_shape=jax.ShapeDtypeStruct((B, S, D), jnp.float32),
    )(x, w1, b1.reshape(1, DFF), w2, b2.reshape(1, D))

@jax.jit
def kernel(src, Wqk, Wv, Wo, rot, W1, b1, W2, b2):
    qk, v, bkt = k1_proj_hash(src, Wqk, Wv, rot)
    inv = k2_inv(bkt)
    inv_t = inv.transpose(0, 2, 1).reshape(B, H, NH, S)
    order = jnp.argsort(inv_t, axis=-1)
    qk_bh = qk.reshape(B, S, H, DH).transpose(0, 2, 1, 3)
    v_bh = v.reshape(B, S, H, DH).transpose(0, 2, 1, 3)
    qk_s = jnp.take_along_axis(
        jnp.broadcast_to(qk_bh[:, :, None], (B, H, NH, S, DH)),
        order[..., None], axis=3)
    v_s = jnp.take_along_axis(
        jnp.broadcast_to(v_bh[:, :, None], (B, H, NH, S, DH)),
        order[..., None], axis=3)
    o_s, lse_s = k4_attention(qk_s.reshape(NINST, S, DH),
                              v_s.reshape(NINST, S, DH))
    o_s = o_s.reshape(B, H, NH, S, DH)
    lse_s = lse_s.reshape(B, H, NH, S, 16)
    o_u = jnp.take_along_axis(o_s, inv_t[..., None], axis=3)
    lse_u = jnp.take_along_axis(lse_s, inv_t[..., None], axis=3)
    o_u = o_u.transpose(2, 0, 3, 1, 4).reshape(NH, B, S, D)
    lse_u = lse_u.transpose(2, 0, 1, 3, 4)
    attn_p = k6_combine_wo(o_u, lse_u, Wo)
    return k7_ffn(attn_p, W1, b1, W2, b2)


# speedup vs baseline: 2.5545x; 1.1720x over previous
import jax
import jax.numpy as jnp
import numpy as np
from jax import lax
from jax.experimental import pallas as pl
from jax.experimental.pallas import tpu as pltpu

B, S, D, H = 4, 4096, 1024, 16
DH = D // H
BUCKET = 64
NH = 4
NCH = S // BUCKET
DFF = 4096
NINST = B * H * NH
SCALE = 1.0 / np.sqrt(DH)

BM = 512


def _k1_body(src_ref, wqk_ref, wv_ref, rot_ref, qk_ref, v_ref, bkt_ref):
    x = src_ref[0]
    qk = jnp.dot(x, wqk_ref[...], preferred_element_type=jnp.float32)
    v = jnp.dot(x, wv_ref[...], preferred_element_type=jnp.float32)
    qk_ref[0] = qk
    v_ref[0] = v
    rot2 = rot_ref[...]
    iota = lax.broadcasted_iota(jnp.int32, (BM, 2 * 32), 1)
    parts = []
    for h in range(H):
        qh = qk[:, h * DH:(h + 1) * DH]
        n = jnp.sqrt(jnp.sum(qh * qh, axis=1, keepdims=True))
        qn = qh / (n + 1e-6)
        rh = jnp.dot(qn, rot2, preferred_element_type=jnp.float32)
        for r in range(NH):
            vals = rh[:, r * 32:(r + 1) * 32]
            cat = jnp.concatenate([vals, -vals], axis=1)
            m = jnp.max(cat, axis=1, keepdims=True)
            idx = jnp.min(jnp.where(cat >= m, iota, 2 * 32), axis=1,
                          keepdims=True)
            parts.append(idx)
    bkt_ref[0] = jnp.concatenate(parts, axis=1)


def k1_proj_hash(src, Wqk, Wv, rot):
    rot2 = rot.reshape(DH, NH * 32)
    return pl.pallas_call(
        _k1_body,
        grid=(B, S // BM),
        in_specs=[
            pl.BlockSpec((1, BM, D), lambda b, s: (b, s, 0)),
            pl.BlockSpec((D, D), lambda b, s: (0, 0)),
            pl.BlockSpec((D, D), lambda b, s: (0, 0)),
            pl.BlockSpec((DH, NH * 32), lambda b, s: (0, 0)),
        ],
        out_specs=[
            pl.BlockSpec((1, BM, D), lambda b, s: (b, s, 0)),
            pl.BlockSpec((1, BM, D), lambda b, s: (b, s, 0)),
            pl.BlockSpec((1, BM, H * NH), lambda b, s: (b, s, 0)),
        ],
        out_shape=[
            jax.ShapeDtypeStruct((B, S, D), jnp.float32),
            jax.ShapeDtypeStruct((B, S, D), jnp.float32),
            jax.ShapeDtypeStruct((B, S, H * NH), jnp.int32),
        ],
    )(src, Wqk, Wv, rot2)


TS = 128
NT = S // TS


def _k2_body(bkt_ref, inv_ref):
    bkt = bkt_ref[0]
    totals = []
    for beta in range(BUCKET):
        totals.append(jnp.sum((bkt == beta).astype(jnp.float32), axis=0,
                              keepdims=True))
    offs = []
    run = jnp.zeros((1, H * NH), jnp.float32)
    for beta in range(BUCKET):
        offs.append(run)
        run = run + totals[beta]
    ii = lax.broadcasted_iota(jnp.int32, (TS, TS), 0)
    jj = lax.broadcasted_iota(jnp.int32, (TS, TS), 1)
    T = (jj <= ii).astype(jnp.float32)

    def tile_step(t, carrys):
        blk = bkt_ref[0, pl.ds(t * TS, TS), :]
        acc = jnp.zeros((TS, H * NH), jnp.float32)
        new_carrys = []
        for beta in range(BUCKET):
            I = (blk == beta).astype(jnp.float32)
            incl = jnp.dot(T, I, preferred_element_type=jnp.float32)
            acc = acc + I * (incl - 1.0 + carrys[beta] + offs[beta])
            new_carrys.append(carrys[beta] + incl[TS - 1:TS, :])
        inv_ref[0, pl.ds(t * TS, TS), :] = acc.astype(jnp.int32)
        return tuple(new_carrys)

    lax.fori_loop(0, NT, tile_step,
                  tuple(jnp.zeros((1, H * NH), jnp.float32)
                        for _ in range(BUCKET)))


def k2_inv(bkt):
    return pl.pallas_call(
        _k2_body,
        grid=(B,),
        in_specs=[pl.BlockSpec((1, S, H * NH), lambda b: (b, 0, 0))],
        out_specs=pl.BlockSpec((1, S, H * NH), lambda b: (b, 0, 0)),
        out_shape=jax.ShapeDtypeStruct((B, S, H * NH), jnp.int32),
    )(bkt)


def _k4_body(qk_ref, v_ref, o_ref, lse_ref, kn_ref):
    qk = qk_ref[0]
    n = jnp.sqrt(jnp.sum(qk * qk, axis=1, keepdims=True))
    kn_ref[...] = qk * (1.0 / (n + 1e-6))
    qi = lax.broadcasted_iota(jnp.int32, (BUCKET, 2 * BUCKET), 0)
    ki = lax.broadcasted_iota(jnp.int32, (BUCKET, 2 * BUCKET), 1)
    selfmask = qi == ki

    def chunk(c, _):
        p = (c + NCH - 1) % NCH
        qc = qk_ref[0, pl.ds(c * BUCKET, BUCKET), :]
        kcat = jnp.concatenate(
            [kn_ref[pl.ds(c * BUCKET, BUCKET), :],
             kn_ref[pl.ds(p * BUCKET, BUCKET), :]], axis=0)
        vcat = jnp.concatenate(
            [v_ref[0, pl.ds(c * BUCKET, BUCKET), :],
             v_ref[0, pl.ds(p * BUCKET, BUCKET), :]], axis=0)
        dots = lax.dot_general(qc, kcat, (((1,), (1,)), ((), ())),
                               preferred_element_type=jnp.float32) * SCALE
        dots = jnp.where(selfmask, -1e5, dots)
        m = jnp.max(dots, axis=1, keepdims=True)
        p_ = jnp.exp(dots - m)
        s = jnp.sum(p_, axis=1, keepdims=True)
        lse = m + jnp.log(s)
        o = jnp.dot(p_, vcat, preferred_element_type=jnp.float32) * (1.0 / s)
        o_ref[0, pl.ds(c * BUCKET, BUCKET), :] = o
        lse_ref[0, pl.ds(c * BUCKET, BUCKET), :] = jnp.broadcast_to(
            lse, (BUCKET, 16))
        return 0

    lax.fori_loop(0, NCH, chunk, 0)


def k4_attention(qk_s, v_s):
    return pl.pallas_call(
        _k4_body,
        grid=(NINST,),
        in_specs=[
            pl.BlockSpec((1, S, DH), lambda i: (i, 0, 0)),
            pl.BlockSpec((1, S, DH), lambda i: (i, 0, 0)),
        ],
        out_specs=[
            pl.BlockSpec((1, S, DH), lambda i: (i, 0, 0)),
            pl.BlockSpec((1, S, 16), lambda i: (i, 0, 0)),
        ],
        out_shape=[
            jax.ShapeDtypeStruct((NINST, S, DH), jnp.float32),
            jax.ShapeDtypeStruct((NINST, S, 16), jnp.float32),
        ],
        scratch_shapes=[pltpu.VMEM((S, DH), jnp.float32)],
    )(qk_s, v_s)


BMC = 128


def _k6_body(o_ref, lse_ref, wo_ref, out_ref):
    ls = [lse_ref[r, 0] for r in range(NH)]
    m = ls[0]
    for r in range(1, NH):
        m = jnp.maximum(m, ls[r])
    es = [jnp.exp(l - m) for l in ls]
    ssum = es[0]
    for r in range(1, NH):
        ssum = ssum + es[r]
    bmat = jnp.full((16, DH), 1.0 / 16.0, jnp.float32)
    parts = []
    for h in range(H):
        acc = jnp.zeros((BMC, DH), jnp.float32)
        for r in range(NH):
            w = jnp.dot(es[r][h] * (1.0 / ssum[h]), bmat,
                        preferred_element_type=jnp.float32)
            acc = acc + o_ref[r, 0, :, h * DH:(h + 1) * DH] * w
        parts.append(acc)
    attn = jnp.concatenate(parts, axis=1)
    out_ref[0] = jnp.dot(attn, wo_ref[...], preferred_element_type=jnp.float32)


def k6_combine_wo(o_u, lse_u, Wo):
    return pl.pallas_call(
        _k6_body,
        grid=(B, S // BMC),
        in_specs=[
            pl.BlockSpec((NH, 1, BMC, D), lambda b, s: (0, b, s, 0)),
            pl.BlockSpec((NH, 1, H, BMC, 16), lambda b, s: (0, b, 0, s, 0)),
            pl.BlockSpec((D, D), lambda b, s: (0, 0)),
        ],
        out_specs=pl.BlockSpec((1, BMC, D), lambda b, s: (b, s, 0)),
        out_shape=jax.ShapeDtypeStruct((B, S, D), jnp.float32),
    )(o_u, lse_u, Wo)


FT = 1024


def _k7_body(x_ref, w1_ref, b1_ref, w2_ref, b2_ref, o_ref):
    t = pl.program_id(2)
    h = jnp.dot(x_ref[0], w1_ref[...], preferred_element_type=jnp.float32)
    h = jnp.maximum(h + b1_ref[...], 0.0)
    part = jnp.dot(h, w2_ref[...], preferred_element_type=jnp.float32)

    @pl.when(t == 0)
    def _():
        o_ref[0] = part + b2_ref[...]

    @pl.when(t != 0)
    def _():
        o_ref[0] = o_ref[0] + part


def k7_ffn(x, w1, b1, w2, b2):
    return pl.pallas_call(
        _k7_body,
        grid=(B, S // BM, DFF // FT),
        in_specs=[
            pl.BlockSpec((1, BM, D), lambda b, s, t: (b, s, 0)),
            pl.BlockSpec((D, FT), lambda b, s, t: (0, t)),
            pl.BlockSpec((1, FT), lambda b, s, t: (0, t)),
            pl.BlockSpec((FT, D), lambda b, s, t: (t, 0)),
            pl.BlockSpec((1, D), lambda b, s, t: (0, 0)),
        ],
        out_specs=pl.BlockSpec((1, BM, D), lambda b, s, t: (b, s, 0)),
        out_shape=jax.ShapeDtypeStruct((B, S, D), jnp.float32),
    )(x, w1, b1.reshape(1, DFF), w2, b2.reshape(1, D))

from jax.experimental.pallas import tpu_sc as plsc
import functools

NW = 32
IPW = NINST // NW
SR = S // 128
QR = 4
NQ = SR // QR

_mesh = plsc.VectorSubcoreMesh(core_axis_name="c", subcore_axis_name="s")


def _k3_body(inv_hbm, qk4, v4, qk_s4, v_s4, inv_v, src_v, qbuf, vbuf, sem):
    wid = lax.axis_index("s") * 2 + lax.axis_index("c")

    def inst_body(k, carry):
        inst = wid * IPW + k
        b = inst // (H * NH)
        c = inst % (H * NH)
        h = c // NH
        pltpu.sync_copy(inv_hbm.at[inst], inv_v)

        def mkidx(rr, carry2):
            for j in range(8):
                i0 = rr * 128 + j * 16
                src_v[rr, pl.ds(j * 16, 16)] = (
                    (lax.iota(jnp.int32, 16) + i0) * H + h)
            return carry2

        lax.fori_loop(0, SR, mkidx, 0)
        for q in range(NQ):
            cps = []
            for j in range(QR):
                g = q * QR + j
                cps.append(pltpu.async_copy(
                    qk4.at[b].at[src_v.at[g]],
                    qbuf.at[pl.ds(j * 128, 128)], sem))
                cps.append(pltpu.async_copy(
                    v4.at[b].at[src_v.at[g]],
                    vbuf.at[pl.ds(j * 128, 128)], sem))
            for cp in cps:
                cp.wait()
            cps = []
            for j in range(QR):
                g = q * QR + j
                cps.append(pltpu.async_copy(
                    qbuf.at[pl.ds(j * 128, 128)],
                    qk_s4.at[inst].at[inv_v.at[g]], sem))
                cps.append(pltpu.async_copy(
                    vbuf.at[pl.ds(j * 128, 128)],
                    v_s4.at[inst].at[inv_v.at[g]], sem))
            for cp in cps:
                cp.wait()
        return carry

    lax.fori_loop(0, IPW, inst_body, 0)


def k3_sort_gather(inv2, qk, v):
    qk4 = qk.reshape(B, S * H, DH)
    v4 = v.reshape(B, S * H, DH)
    f = pl.kernel(
        _k3_body,
        mesh=_mesh,
        compiler_params=pltpu.CompilerParams(use_tc_tiling_on_sc=False),
        out_type=[
            jax.ShapeDtypeStruct((NINST, S, DH), jnp.float32),
            jax.ShapeDtypeStruct((NINST, S, DH), jnp.float32),
        ],
        scratch_types=[
            pltpu.VMEM((SR, 128), jnp.int32),
            pltpu.VMEM((SR, 128), jnp.int32),
            pltpu.VMEM((QR * 128, DH), jnp.float32),
            pltpu.VMEM((QR * 128, DH), jnp.float32),
            pltpu.SemaphoreType.DMA,
        ],
    )
    return f(inv2, qk4, v4)


def _k5_body(inv_hbm, o_s3, lse_s3, o_u3, lse_u2,
             inv_v, dst_v, obuf, lbuf, sem):
    wid = lax.axis_index("s") * 2 + lax.axis_index("c")

    def inst_body(k, carry):
        inst = wid * IPW + k
        b = inst // (H * NH)
        c = inst % (H * NH)
        h = c // NH
        r = c % NH
        rb = r * B + b
        base_l = ((rb * H) + h) * S
        pltpu.sync_copy(inv_hbm.at[inst], inv_v)

        def mkidx(rr, carry2):
            for j in range(8):
                i0 = rr * 128 + j * 16
                dst_v[rr, pl.ds(j * 16, 16)] = (
                    (lax.iota(jnp.int32, 16) + i0) * H + h)
            return carry2

        lax.fori_loop(0, SR, mkidx, 0)
        for q in range(NQ):
            cps = []
            for j in range(QR):
                g = q * QR + j
                cps.append(pltpu.async_copy(
                    o_s3.at[inst].at[inv_v.at[g]],
                    obuf.at[pl.ds(j * 128, 128)], sem))
                cps.append(pltpu.async_copy(
                    lse_s3.at[inst].at[inv_v.at[g]],
                    lbuf.at[pl.ds(j * 128, 128)], sem))
            for cp in cps:
                cp.wait()
            cps = []
            for j in range(QR):
                g = q * QR + j
                cps.append(pltpu.async_copy(
                    obuf.at[pl.ds(j * 128, 128)],
                    o_u3.at[rb].at[dst_v.at[g]], sem))
            cps.append(pltpu.async_copy(
                lbuf, lse_u2.at[pl.ds(base_l + q * QR * 128, QR * 128)],
                sem))
            for cp in cps:
                cp.wait()
        return carry

    lax.fori_loop(0, IPW, inst_body, 0)


def k5_unsort_scatter(inv2, o_s, lse_s):
    f = pl.kernel(
        _k5_body,
        mesh=_mesh,
        compiler_params=pltpu.CompilerParams(use_tc_tiling_on_sc=False),
        out_type=[
            jax.ShapeDtypeStruct((NH * B, S * H, DH), jnp.float32),
            jax.ShapeDtypeStruct((NH * B * H * S, 16), jnp.float32),
        ],
        scratch_types=[
            pltpu.VMEM((SR, 128), jnp.int32),
            pltpu.VMEM((SR, 128), jnp.int32),
            pltpu.VMEM((QR * 128, DH), jnp.float32),
            pltpu.VMEM((QR * 128, 16), jnp.float32),
            pltpu.SemaphoreType.DMA,
        ],
    )
    return f(inv2, o_s, lse_s)


@jax.jit
def kernel(src, Wqk, Wv, Wo, rot, W1, b1, W2, b2):
    qk, v, bkt = k1_proj_hash(src, Wqk, Wv, rot)
    inv = k2_inv(bkt)
    inv2 = inv.transpose(0, 2, 1).reshape(NINST, SR, 128)
    qk_s, v_s = k3_sort_gather(inv2, qk, v)
    o_s, lse_s = k4_attention(qk_s, v_s)
    o_u_tab, lse_u_tab = k5_unsort_scatter(inv2, o_s, lse_s)
    o_u = o_u_tab.reshape(NH, B, S, D)
    lse_u = lse_u_tab.reshape(NH, B, H, S, 16)
    attn_p = k6_combine_wo(o_u, lse_u, Wo)
    return k7_ffn(attn_p, W1, b1, W2, b2)


# speedup vs baseline: 3.5213x; 1.3785x over previous
import jax
import jax.numpy as jnp
import numpy as np
from jax import lax
from jax.experimental import pallas as pl
from jax.experimental.pallas import tpu as pltpu

B, S, D, H = 4, 4096, 1024, 16
DH = D // H
BUCKET = 64
NH = 4
NCH = S // BUCKET
DFF = 4096
NINST = B * H * NH
SCALE = 1.0 / np.sqrt(DH)

BM = 512


def _k1_body(src_ref, wqk_ref, wv_ref, rot_ref, qk_ref, v_ref, bkt_ref):
    x = src_ref[0]
    qk = jnp.dot(x, wqk_ref[...], preferred_element_type=jnp.float32)
    v = jnp.dot(x, wv_ref[...], preferred_element_type=jnp.float32)
    qk_ref[0] = qk.astype(jnp.bfloat16)
    v_ref[0] = v.astype(jnp.bfloat16)
    rot2 = rot_ref[...]
    iota = lax.broadcasted_iota(jnp.int32, (BM, 2 * 32), 1)
    parts = []
    for h in range(H):
        qh = qk[:, h * DH:(h + 1) * DH]
        n = jnp.sqrt(jnp.sum(qh * qh, axis=1, keepdims=True))
        qn = qh / (n + 1e-6)
        rh = jnp.dot(qn, rot2, preferred_element_type=jnp.float32)
        for r in range(NH):
            vals = rh[:, r * 32:(r + 1) * 32]
            cat = jnp.concatenate([vals, -vals], axis=1)
            m = jnp.max(cat, axis=1, keepdims=True)
            idx = jnp.min(jnp.where(cat >= m, iota, 2 * 32), axis=1,
                          keepdims=True)
            parts.append(idx)
    bkt_ref[0] = jnp.concatenate(parts, axis=1)


def k1_proj_hash(src, Wqk, Wv, rot):
    rot2 = rot.reshape(DH, NH * 32)
    return pl.pallas_call(
        _k1_body,
        grid=(B, S // BM),
        in_specs=[
            pl.BlockSpec((1, BM, D), lambda b, s: (b, s, 0)),
            pl.BlockSpec((D, D), lambda b, s: (0, 0)),
            pl.BlockSpec((D, D), lambda b, s: (0, 0)),
            pl.BlockSpec((DH, NH * 32), lambda b, s: (0, 0)),
        ],
        out_specs=[
            pl.BlockSpec((1, BM, D), lambda b, s: (b, s, 0)),
            pl.BlockSpec((1, BM, D), lambda b, s: (b, s, 0)),
            pl.BlockSpec((1, BM, H * NH), lambda b, s: (b, s, 0)),
        ],
        out_shape=[
            jax.ShapeDtypeStruct((B, S, D), jnp.bfloat16),
            jax.ShapeDtypeStruct((B, S, D), jnp.bfloat16),
            jax.ShapeDtypeStruct((B, S, H * NH), jnp.int32),
        ],
    )(src, Wqk, Wv, rot2)


TS = 128
NT = S // TS


def _k2_body(bkt_ref, inv_ref):
    bkt = bkt_ref[0]
    totals = []
    for beta in range(BUCKET):
        totals.append(jnp.sum((bkt == beta).astype(jnp.float32), axis=0,
                              keepdims=True))
    offs = []
    run = jnp.zeros((1, H * NH), jnp.float32)
    for beta in range(BUCKET):
        offs.append(run)
        run = run + totals[beta]
    ii = lax.broadcasted_iota(jnp.int32, (TS, TS), 0)
    jj = lax.broadcasted_iota(jnp.int32, (TS, TS), 1)
    T = (jj <= ii).astype(jnp.float32)

    def tile_step(t, carrys):
        blk = bkt_ref[0, pl.ds(t * TS, TS), :]
        acc = jnp.zeros((TS, H * NH), jnp.float32)
        new_carrys = []
        for beta in range(BUCKET):
            I = (blk == beta).astype(jnp.float32)
            incl = jnp.dot(T, I, preferred_element_type=jnp.float32)
            acc = acc + I * (incl - 1.0 + carrys[beta] + offs[beta])
            new_carrys.append(carrys[beta] + incl[TS - 1:TS, :])
        inv_ref[0, pl.ds(t * TS, TS), :] = acc.astype(jnp.int32)
        return tuple(new_carrys)

    lax.fori_loop(0, NT, tile_step,
                  tuple(jnp.zeros((1, H * NH), jnp.float32)
                        for _ in range(BUCKET)))


def k2_inv(bkt):
    return pl.pallas_call(
        _k2_body,
        grid=(B,),
        in_specs=[pl.BlockSpec((1, S, H * NH), lambda b: (b, 0, 0))],
        out_specs=pl.BlockSpec((1, S, H * NH), lambda b: (b, 0, 0)),
        out_shape=jax.ShapeDtypeStruct((B, S, H * NH), jnp.int32),
    )(bkt)


CG = 4
QB = CG * BUCKET
KB = QB + BUCKET
NB4 = NCH // CG


def _k4_body(qk_ref, v_ref, o_ref, lse_ref, kn_ref):
    qk = qk_ref[0].astype(jnp.float32)
    n = jnp.sqrt(jnp.sum(qk * qk, axis=1, keepdims=True))
    kn_ref[...] = (qk * (1.0 / (n + 1e-6))).astype(jnp.bfloat16)
    ri = lax.broadcasted_iota(jnp.int32, (QB, KB), 0)
    ci = lax.broadcasted_iota(jnp.int32, (QB, KB), 1)
    band = (ci // BUCKET == ri // BUCKET) | (ci // BUCKET == ri // BUCKET + 1)
    dead = (~band) | (ci == ri + BUCKET)

    def blk(cb, _):
        p0 = ((cb * CG - 1) % NCH) * BUCKET
        qc = qk_ref[0, pl.ds(cb * QB, QB), :]
        kcat = jnp.concatenate(
            [kn_ref[pl.ds(p0, BUCKET), :],
             kn_ref[pl.ds(cb * QB, QB), :]], axis=0)
        vcat = jnp.concatenate(
            [v_ref[0, pl.ds(p0, BUCKET), :],
             v_ref[0, pl.ds(cb * QB, QB), :]], axis=0)
        dots = lax.dot_general(qc, kcat, (((1,), (1,)), ((), ())),
                               preferred_element_type=jnp.float32) * SCALE
        dots = jnp.where(dead, -1e5, dots)
        m = jnp.max(dots, axis=1, keepdims=True)
        p_ = jnp.exp(dots - m)
        s = jnp.sum(p_, axis=1, keepdims=True)
        lse = m + jnp.log(s)
        o = jnp.dot(p_.astype(jnp.bfloat16), vcat,
                    preferred_element_type=jnp.float32) * (1.0 / s)
        o_ref[0, pl.ds(cb * QB, QB), :] = o.astype(jnp.bfloat16)
        lse_ref[0, pl.ds(cb * QB, QB), :] = jnp.broadcast_to(lse, (QB, 16))
        return 0

    lax.fori_loop(0, NB4, blk, 0)


def k4_attention(qk_s, v_s):
    return pl.pallas_call(
        _k4_body,
        grid=(NINST,),
        in_specs=[
            pl.BlockSpec((1, S, DH), lambda i: (i, 0, 0)),
            pl.BlockSpec((1, S, DH), lambda i: (i, 0, 0)),
        ],
        out_specs=[
            pl.BlockSpec((1, S, DH), lambda i: (i, 0, 0)),
            pl.BlockSpec((1, S, 16), lambda i: (i, 0, 0)),
        ],
        out_shape=[
            jax.ShapeDtypeStruct((NINST, S, DH), jnp.bfloat16),
            jax.ShapeDtypeStruct((NINST, S, 16), jnp.float32),
        ],
        scratch_shapes=[pltpu.VMEM((S, DH), jnp.bfloat16)],
    )(qk_s, v_s)


BMC = 128


def _k6_body(o_ref, lse_ref, wo_ref, out_ref):
    ls = [lse_ref[r, 0] for r in range(NH)]
    m = ls[0]
    for r in range(1, NH):
        m = jnp.maximum(m, ls[r])
    es = [jnp.exp(l - m) for l in ls]
    ssum = es[0]
    for r in range(1, NH):
        ssum = ssum + es[r]
    bmat = jnp.full((16, DH), 1.0 / 16.0, jnp.float32)
    parts = []
    for h in range(H):
        acc = jnp.zeros((BMC, DH), jnp.float32)
        for r in range(NH):
            w = jnp.dot(es[r][h] * (1.0 / ssum[h]), bmat,
                        preferred_element_type=jnp.float32)
            acc = acc + o_ref[r, 0, :, h * DH:(h + 1) * DH].astype(
                jnp.float32) * w
        parts.append(acc)
    attn = jnp.concatenate(parts, axis=1)
    out_ref[0] = jnp.dot(attn, wo_ref[...], preferred_element_type=jnp.float32)


def k6_combine_wo(o_u, lse_u, Wo):
    return pl.pallas_call(
        _k6_body,
        grid=(B, S // BMC),
        in_specs=[
            pl.BlockSpec((NH, 1, BMC, D), lambda b, s: (0, b, s, 0)),
            pl.BlockSpec((NH, 1, H, BMC, 16), lambda b, s: (0, b, 0, s, 0)),
            pl.BlockSpec((D, D), lambda b, s: (0, 0)),
        ],
        out_specs=pl.BlockSpec((1, BMC, D), lambda b, s: (b, s, 0)),
        out_shape=jax.ShapeDtypeStruct((B, S, D), jnp.float32),
    )(o_u, lse_u, Wo)


FT = 1024


def _k7_body(x_ref, w1_ref, b1_ref, w2_ref, b2_ref, o_ref):
    t = pl.program_id(2)
    h = jnp.dot(x_ref[0], w1_ref[...], preferred_element_type=jnp.float32)
    h = jnp.maximum(h + b1_ref[...], 0.0)
    part = jnp.dot(h, w2_ref[...], preferred_element_type=jnp.float32)

    @pl.when(t == 0)
    def _():
        o_ref[0] = part + b2_ref[...]

    @pl.when(t != 0)
    def _():
        o_ref[0] = o_ref[0] + part


def k7_ffn(x, w1, b1, w2, b2):
    return pl.pallas_call(
        _k7_body,
        grid=(B, S // BM, DFF // FT),
        in_specs=[
            pl.BlockSpec((1, BM, D), lambda b, s, t: (b, s, 0)),
            pl.BlockSpec((D, FT), lambda b, s, t: (0, t)),
            pl.BlockSpec((1, FT), lambda b, s, t: (0, t)),
            pl.BlockSpec((FT, D), lambda b, s, t: (t, 0)),
            pl.BlockSpec((1, D), lambda b, s, t: (0, 0)),
        ],
        out_specs=pl.BlockSpec((1, BM, D), lambda b, s, t: (b, s, 0)),
        out_shape=jax.ShapeDtypeStruct((B, S, D), jnp.float32),
    )(x, w1, b1.reshape(1, DFF), w2, b2.reshape(1, D))

from jax.experimental.pallas import tpu_sc as plsc
import functools

NW = 32
IPW = NINST // NW
SR = S // 128
QR = 4
NQ = SR // QR

def _sc_mesh():
    return plsc.VectorSubcoreMesh(core_axis_name="c", subcore_axis_name="s")


def _k3_body(inv_hbm, qk4, v4, qk_s4, v_s4, inv_v, src_v, qbuf, vbuf, sem):
    wid = lax.axis_index("s") * 2 + lax.axis_index("c")

    def inst_body(k, carry):
        inst = wid * IPW + k
        b = inst // (H * NH)
        c = inst % (H * NH)
        h = c // NH
        pltpu.sync_copy(inv_hbm.at[inst], inv_v)

        def mkidx(rr, carry2):
            for j in range(8):
                i0 = rr * 128 + j * 16
                src_v[rr, pl.ds(j * 16, 16)] = (
                    (lax.iota(jnp.int32, 16) + i0) * H + h)
            return carry2

        lax.fori_loop(0, SR, mkidx, 0)
        for q in range(NQ):
            cps = []
            for j in range(QR):
                g = q * QR + j
                cps.append(pltpu.async_copy(
                    qk4.at[b].at[src_v.at[g]],
                    qbuf.at[pl.ds(j * 128, 128)], sem))
                cps.append(pltpu.async_copy(
                    v4.at[b].at[src_v.at[g]],
                    vbuf.at[pl.ds(j * 128, 128)], sem))
            for cp in cps:
                cp.wait()
            cps = []
            for j in range(QR):
                g = q * QR + j
                cps.append(pltpu.async_copy(
                    qbuf.at[pl.ds(j * 128, 128)],
                    qk_s4.at[inst].at[inv_v.at[g]], sem))
                cps.append(pltpu.async_copy(
                    vbuf.at[pl.ds(j * 128, 128)],
                    v_s4.at[inst].at[inv_v.at[g]], sem))
            for cp in cps:
                cp.wait()
        return carry

    lax.fori_loop(0, IPW, inst_body, 0)


def k3_sort_gather(inv2, qk, v):
    qk4 = qk.reshape(B, S * H, DH)
    v4 = v.reshape(B, S * H, DH)
    f = pl.kernel(
        _k3_body,
        mesh=_sc_mesh(),
        compiler_params=pltpu.CompilerParams(use_tc_tiling_on_sc=False),
        out_type=[
            jax.ShapeDtypeStruct((NINST, S, DH), jnp.bfloat16),
            jax.ShapeDtypeStruct((NINST, S, DH), jnp.bfloat16),
        ],
        scratch_types=[
            pltpu.VMEM((SR, 128), jnp.int32),
            pltpu.VMEM((SR, 128), jnp.int32),
            pltpu.VMEM((QR * 128, DH), jnp.bfloat16),
            pltpu.VMEM((QR * 128, DH), jnp.bfloat16),
            pltpu.SemaphoreType.DMA,
        ],
    )
    return f(inv2, qk4, v4)


def _k5_body(inv_hbm, o_s3, lse_s3, o_u3, lse_u2,
             inv_v, dst_v, obuf, lbuf, sem):
    wid = lax.axis_index("s") * 2 + lax.axis_index("c")

    def inst_body(k, carry):
        inst = wid * IPW + k
        b = inst // (H * NH)
        c = inst % (H * NH)
        h = c // NH
        r = c % NH
        rb = r * B + b
        base_l = ((rb * H) + h) * S
        pltpu.sync_copy(inv_hbm.at[inst], inv_v)

        def mkidx(rr, carry2):
            for j in range(8):
                i0 = rr * 128 + j * 16
                dst_v[rr, pl.ds(j * 16, 16)] = (
                    (lax.iota(jnp.int32, 16) + i0) * H + h)
            return carry2

        lax.fori_loop(0, SR, mkidx, 0)
        for q in range(NQ):
            cps = []
            for j in range(QR):
                g = q * QR + j
                cps.append(pltpu.async_copy(
                    o_s3.at[inst].at[inv_v.at[g]],
                    obuf.at[pl.ds(j * 128, 128)], sem))
                cps.append(pltpu.async_copy(
                    lse_s3.at[inst].at[inv_v.at[g]],
                    lbuf.at[pl.ds(j * 128, 128)], sem))
            for cp in cps:
                cp.wait()
            cps = []
            for j in range(QR):
                g = q * QR + j
                cps.append(pltpu.async_copy(
                    obuf.at[pl.ds(j * 128, 128)],
                    o_u3.at[rb].at[dst_v.at[g]], sem))
            cps.append(pltpu.async_copy(
                lbuf, lse_u2.at[pl.ds(base_l + q * QR * 128, QR * 128)],
                sem))
            for cp in cps:
                cp.wait()
        return carry

    lax.fori_loop(0, IPW, inst_body, 0)


def k5_unsort_scatter(inv2, o_s, lse_s):
    f = pl.kernel(
        _k5_body,
        mesh=_sc_mesh(),
        compiler_params=pltpu.CompilerParams(use_tc_tiling_on_sc=False),
        out_type=[
            jax.ShapeDtypeStruct((NH * B, S * H, DH), jnp.bfloat16),
            jax.ShapeDtypeStruct((NH * B * H * S, 16), jnp.float32),
        ],
        scratch_types=[
            pltpu.VMEM((SR, 128), jnp.int32),
            pltpu.VMEM((SR, 128), jnp.int32),
            pltpu.VMEM((QR * 128, DH), jnp.bfloat16),
            pltpu.VMEM((QR * 128, 16), jnp.float32),
            pltpu.SemaphoreType.DMA,
        ],
    )
    return f(inv2, o_s, lse_s)


@jax.jit
def kernel(src, Wqk, Wv, Wo, rot, W1, b1, W2, b2):
    qk, v, bkt = k1_proj_hash(src, Wqk, Wv, rot)
    inv = k2_inv(bkt)
    inv2 = inv.transpose(0, 2, 1).reshape(NINST, SR, 128)
    qk_s, v_s = k3_sort_gather(inv2, qk, v)
    o_s, lse_s = k4_attention(qk_s, v_s)
    o_u_tab, lse_u_tab = k5_unsort_scatter(inv2, o_s, lse_s)
    o_u = o_u_tab.reshape(NH, B, S, D)
    lse_u = lse_u_tab.reshape(NH, B, H, S, 16)
    attn_p = k6_combine_wo(o_u, lse_u, Wo)
    return k7_ffn(attn_p, W1, b1, W2, b2)
